# baseline (speedup 1.0000x reference)
import functools

import jax
import jax.numpy as jnp
from jax import lax
from jax.experimental import pallas as pl
from jax.experimental.pallas import tpu as pltpu
from jax.experimental.pallas import tpu_sc as plsc

_info = plsc.get_sparse_core_info()
_NC, _NS = _info.num_cores, _info.num_subcores
_NW = _NC * _NS

_D = 64
_NH = 50

_mesh = plsc.VectorSubcoreMesh(core_axis_name="c", subcore_axis_name="s")
_params = pltpu.CompilerParams(
    use_tc_tiling_on_sc=True, needs_layout_passes=False
)

_V = 1000000
_WBLK = 128
_NFULL = _V // _WBLK
_VREM = _V - _NFULL * _WBLK
_UPT = (_NFULL + 1 + _NW - 1) // _NW


@functools.partial(
    pl.kernel,
    mesh=_mesh,
    out_type=jax.ShapeDtypeStruct((_NH, _D, 4096), jnp.float32),
    scratch_types=[
        pltpu.VMEM((3, 128), jnp.int32),
        pltpu.VMEM((3, 128), jnp.int32),
        pltpu.VMEM((3, 128), jnp.int32),
        pltpu.VMEM((3, 128, 128), jnp.float32),
        pltpu.VMEM((3, _D, 128), jnp.float32),
        pltpu.SemaphoreType.DMA,
        pltpu.SemaphoreType.DMA,
        pltpu.SemaphoreType.DMA,
        pltpu.SemaphoreType.DMA,
        pltpu.SemaphoreType.DMA,
        pltpu.SemaphoreType.DMA,
        pltpu.SemaphoreType.DMA,
        pltpu.SemaphoreType.DMA,
        pltpu.SemaphoreType.DMA,
    ],
    compiler_params=_params,
)
def _lookup(w_rm_hbm, tt_hbm, ot_hbm, tok_v, q_v, hb_v, g_v, o_v,
            ts0, ts1, ts2, gs0, gs1, gs2, os0, os1, os2):
    wid = lax.axis_index("s") * _NC + lax.axis_index("c")
    b0 = pl.multiple_of(wid * 128, 128)
    ts = (ts0, ts1, ts2)
    gs = (gs0, gs1, gs2)
    os = (os0, os1, os2)
    it = lax.iota(jnp.int32, 16)
    jrows = [it + 16 * v for v in range(8)]

    def start_tok(h, k):
        pltpu.async_copy(tt_hbm.at[h, pl.ds(b0, 128)], tok_v.at[k], ts[k])

    def prep(k):
        pltpu.make_async_copy(
            tt_hbm.at[0, pl.ds(0, 128)], tok_v.at[k], ts[k]
        ).wait()
        for v in range(8):
            t = tok_v[k, pl.ds(16 * v, 16)]
            q_v[k, pl.ds(16 * v, 16)] = lax.shift_right_logical(t, 1)
            hb_v[k, pl.ds(16 * v, 16)] = (t & 1) * 64
        pltpu.async_copy(w_rm_hbm.at[q_v.at[k]], g_v.at[k], gs[k])

    def wait_gather(k):
        pltpu.make_async_copy(
            w_rm_hbm.at[q_v.at[k]], g_v.at[k], gs[k]
        ).wait()

    def start_out(h, k):
        pltpu.async_copy(o_v.at[k], ot_hbm.at[h, :, pl.ds(b0, 128)], os[k])

    def wait_out(k):
        pltpu.make_async_copy(
            o_v.at[k], ot_hbm.at[0, :, pl.ds(0, 128)], os[k]
        ).wait()

    rot = [(it + d) & 15 for d in range(16)]

    def permute(k):
        hv = [hb_v[k, pl.ds(16 * v, 16)] for v in range(8)]

        def dbody(d, carry):
            rotd = (it + d) & 15
            crow = [rotd + 16 * cb for cb in range(_D // 16)]
            for jb in range(8):
                t1 = hv[jb] + rotd
                for cb in range(_D // 16):
                    g = plsc.load_gather(
                        g_v.at[k], [jrows[jb], t1 + 16 * cb]
                    )
                    plsc.store_scatter(
                        o_v.at[k], [crow[cb], jrows[jb]], g
                    )
            return carry

        lax.fori_loop(0, 16, dbody, 0)

    for k in range(3):
        start_tok(k, k)
    for k in range(3):
        prep(k)
        start_tok(k + 3, k)

    def body(i, carry):
        for k in range(3):
            u = 3 * i + k

            @pl.when(u < _NH)
            def _():
                wait_gather(k)

                @pl.when(i > 0)
                def _():
                    wait_out(k)

                permute(k)
                start_out(u, k)

                @pl.when(u + 6 < _NH)
                def _():
                    start_tok(u + 6, k)

                @pl.when(u + 3 < _NH)
                def _():
                    prep(k)

        return carry

    lax.fori_loop(0, (_NH + 2) // 3, body, 0)
    for k in range(3):
        wait_out(k)


def kernel(tokens, weights):
    V, D = weights.shape
    w_rm = weights.reshape(V // 2, 2 * D)
    ot = _lookup(w_rm, tokens.T.astype(jnp.int32))
    return ot.transpose(2, 0, 1)


# speedup vs baseline: 1.0036x; 1.0036x over previous
import functools

import jax
import jax.numpy as jnp
from jax import lax
from jax.experimental import pallas as pl
from jax.experimental.pallas import tpu as pltpu
from jax.experimental.pallas import tpu_sc as plsc

_info = plsc.get_sparse_core_info()
_NC, _NS = _info.num_cores, _info.num_subcores
_NW = _NC * _NS

_D = 64
_NH = 50

_mesh = plsc.VectorSubcoreMesh(core_axis_name="c", subcore_axis_name="s")
_params = pltpu.CompilerParams(
    use_tc_tiling_on_sc=True, needs_layout_passes=False
)

_V = 1000000
_WBLK = 128
_NFULL = _V // _WBLK
_VREM = _V - _NFULL * _WBLK
_UPT = (_NFULL + 1 + _NW - 1) // _NW


@functools.partial(
    pl.kernel,
    mesh=_mesh,
    out_type=jax.ShapeDtypeStruct((_NH, _D, 4096), jnp.float32),
    scratch_types=[
        pltpu.VMEM((3, 128), jnp.int32),
        pltpu.VMEM((3, 128), jnp.int32),
        pltpu.VMEM((3, 128), jnp.int32),
        pltpu.VMEM((3, 128, 128), jnp.float32),
        pltpu.VMEM((3, _D, 128), jnp.float32),
        pltpu.SemaphoreType.DMA,
        pltpu.SemaphoreType.DMA,
        pltpu.SemaphoreType.DMA,
        pltpu.SemaphoreType.DMA,
        pltpu.SemaphoreType.DMA,
        pltpu.SemaphoreType.DMA,
        pltpu.SemaphoreType.DMA,
        pltpu.SemaphoreType.DMA,
        pltpu.SemaphoreType.DMA,
    ],
    compiler_params=_params,
)
def _lookup(w_rm_hbm, tt_hbm, ot_hbm, tok_v, q_v, hb_v, g_v, o_v,
            ts0, ts1, ts2, gs0, gs1, gs2, os0, os1, os2):
    wid = lax.axis_index("s") * _NC + lax.axis_index("c")
    b0 = pl.multiple_of(wid * 128, 128)
    ts = (ts0, ts1, ts2)
    gs = (gs0, gs1, gs2)
    os = (os0, os1, os2)
    it = lax.iota(jnp.int32, 16)
    jrows = [it + 16 * v for v in range(8)]

    def start_tok(h, k):
        pltpu.async_copy(tt_hbm.at[h, pl.ds(b0, 128)], tok_v.at[k], ts[k])

    def prep(k):
        pltpu.make_async_copy(
            tt_hbm.at[0, pl.ds(0, 128)], tok_v.at[k], ts[k]
        ).wait()
        for v in range(8):
            t = tok_v[k, pl.ds(16 * v, 16)]
            q_v[k, pl.ds(16 * v, 16)] = lax.shift_right_logical(t, 1)
            hb_v[k, pl.ds(16 * v, 16)] = (t & 1) * 64
        pltpu.async_copy(w_rm_hbm.at[q_v.at[k]], g_v.at[k], gs[k])

    def wait_gather(k):
        pltpu.make_async_copy(
            w_rm_hbm.at[q_v.at[k]], g_v.at[k], gs[k]
        ).wait()

    def start_out(h, k):
        pltpu.async_copy(o_v.at[k], ot_hbm.at[h, :, pl.ds(b0, 128)], os[k])

    def wait_out(k):
        pltpu.make_async_copy(
            o_v.at[k], ot_hbm.at[0, :, pl.ds(0, 128)], os[k]
        ).wait()

    rot = [(it + d) & 15 for d in range(16)]

    def permute(k):
        hv = [hb_v[k, pl.ds(16 * v, 16)] for v in range(8)]

        def dbody(d2, carry):
            for dd in range(2):
                rotd = (it + 2 * d2 + dd) & 15
                crow = [rotd + 16 * cb for cb in range(_D // 16)]
                for jb in range(8):
                    t1 = hv[jb] + rotd
                    for cb in range(_D // 16):
                        g = plsc.load_gather(
                            g_v.at[k], [jrows[jb], t1 + 16 * cb]
                        )
                        plsc.store_scatter(
                            o_v.at[k], [crow[cb], jrows[jb]], g
                        )
            return carry

        lax.fori_loop(0, 8, dbody, 0)

    for k in range(3):
        start_tok(k, k)
    for k in range(3):
        prep(k)
        start_tok(k + 3, k)

    def body(i, carry):
        for k in range(3):
            u = 3 * i + k

            @pl.when(u < _NH)
            def _():
                wait_gather(k)

                @pl.when(i > 0)
                def _():
                    wait_out(k)

                permute(k)
                start_out(u, k)

                @pl.when(u + 6 < _NH)
                def _():
                    start_tok(u + 6, k)

                @pl.when(u + 3 < _NH)
                def _():
                    prep(k)

        return carry

    lax.fori_loop(0, (_NH + 2) // 3, body, 0)
    for k in range(3):
        wait_out(k)


def kernel(tokens, weights):
    V, D = weights.shape
    w_rm = weights.reshape(V // 2, 2 * D)
    ot = _lookup(w_rm, tokens.T.astype(jnp.int32))
    return ot.transpose(2, 0, 1)


# speedup vs baseline: 1.0085x; 1.0048x over previous
import functools

import jax
import jax.numpy as jnp
from jax import lax
from jax.experimental import pallas as pl
from jax.experimental.pallas import tpu as pltpu
from jax.experimental.pallas import tpu_sc as plsc

_info = plsc.get_sparse_core_info()
_NC, _NS = _info.num_cores, _info.num_subcores
_NW = _NC * _NS

_D = 64
_NH = 50

_mesh = plsc.VectorSubcoreMesh(core_axis_name="c", subcore_axis_name="s")
_params = pltpu.CompilerParams(
    use_tc_tiling_on_sc=True, needs_layout_passes=False
)

_V = 1000000
_WBLK = 128
_NFULL = _V // _WBLK
_VREM = _V - _NFULL * _WBLK
_UPT = (_NFULL + 1 + _NW - 1) // _NW


@functools.partial(
    pl.kernel,
    mesh=_mesh,
    out_type=jax.ShapeDtypeStruct((_NH, _D, 4096), jnp.float32),
    scratch_types=[
        pltpu.VMEM((3, 128), jnp.int32),
        pltpu.VMEM((3, 128), jnp.int32),
        pltpu.VMEM((3, 128), jnp.int32),
        pltpu.VMEM((3, 128, 128), jnp.float32),
        pltpu.VMEM((3, _D, 128), jnp.float32),
        pltpu.SemaphoreType.DMA,
        pltpu.SemaphoreType.DMA,
        pltpu.SemaphoreType.DMA,
        pltpu.SemaphoreType.DMA,
        pltpu.SemaphoreType.DMA,
        pltpu.SemaphoreType.DMA,
        pltpu.SemaphoreType.DMA,
        pltpu.SemaphoreType.DMA,
        pltpu.SemaphoreType.DMA,
    ],
    compiler_params=_params,
)
def _lookup(w_rm_hbm, tt_hbm, ot_hbm, tok_v, q_v, hb_v, g_v, o_v,
            ts0, ts1, ts2, gs0, gs1, gs2, os0, os1, os2):
    wid = lax.axis_index("s") * _NC + lax.axis_index("c")
    b0 = pl.multiple_of(wid * 128, 128)
    ts = (ts0, ts1, ts2)
    gs = (gs0, gs1, gs2)
    os = (os0, os1, os2)
    it = lax.iota(jnp.int32, 16)
    jrows = [it + 16 * v for v in range(8)]

    def start_tok(h, k):
        pltpu.async_copy(tt_hbm.at[h, pl.ds(b0, 128)], tok_v.at[k], ts[k])

    def prep(k):
        pltpu.make_async_copy(
            tt_hbm.at[0, pl.ds(0, 128)], tok_v.at[k], ts[k]
        ).wait()
        for v in range(8):
            t = tok_v[k, pl.ds(16 * v, 16)]
            q_v[k, pl.ds(16 * v, 16)] = lax.shift_right_logical(t, 1)
            hb_v[k, pl.ds(16 * v, 16)] = (t & 1) * 64
        pltpu.async_copy(w_rm_hbm.at[q_v.at[k]], g_v.at[k], gs[k])

    def wait_gather(k):
        pltpu.make_async_copy(
            w_rm_hbm.at[q_v.at[k]], g_v.at[k], gs[k]
        ).wait()

    def start_out(h, k):
        pltpu.async_copy(o_v.at[k], ot_hbm.at[h, :, pl.ds(b0, 128)], os[k])

    def wait_out(k):
        pltpu.make_async_copy(
            o_v.at[k], ot_hbm.at[0, :, pl.ds(0, 128)], os[k]
        ).wait()

    rot = [(it + d) & 15 for d in range(16)]

    def permute(k):
        hv = [hb_v[k, pl.ds(16 * v, 16)] for v in range(8)]

        def dbody(d2, carry):
            for dd in range(4):
                rotd = (it + 4 * d2 + dd) & 15
                crow = [rotd + 16 * cb for cb in range(_D // 16)]
                for jb in range(8):
                    t1 = hv[jb] + rotd
                    for cb in range(_D // 16):
                        g = plsc.load_gather(
                            g_v.at[k], [jrows[jb], t1 + 16 * cb]
                        )
                        plsc.store_scatter(
                            o_v.at[k], [crow[cb], jrows[jb]], g
                        )
            return carry

        lax.fori_loop(0, 4, dbody, 0)

    for k in range(3):
        start_tok(k, k)
    for k in range(3):
        prep(k)
        start_tok(k + 3, k)

    def body(i, carry):
        for k in range(3):
            u = 3 * i + k

            @pl.when(u < _NH)
            def _():
                wait_gather(k)

                @pl.when(i > 0)
                def _():
                    wait_out(k)

                permute(k)
                start_out(u, k)

                @pl.when(u + 6 < _NH)
                def _():
                    start_tok(u + 6, k)

                @pl.when(u + 3 < _NH)
                def _():
                    prep(k)

        return carry

    lax.fori_loop(0, (_NH + 2) // 3, body, 0)
    for k in range(3):
        wait_out(k)


def kernel(tokens, weights):
    V, D = weights.shape
    w_rm = weights.reshape(V // 2, 2 * D)
    ot = _lookup(w_rm, tokens.T.astype(jnp.int32))
    return ot.transpose(2, 0, 1)
